# baseline (speedup 1.0000x reference)
import functools
import math

import jax
import jax.numpy as jnp
from jax import lax
from jax.experimental import pallas as pl
from jax.experimental.pallas import tpu as pltpu
from jax.experimental.pallas import tpu_sc as plsc

_VOCAB = 1000000
_D = 64
_DP = 128
_BATCH = 4096
_SEQ = 200
_SCALE = math.sqrt(_D)

_NW = 32
_B_TOT = _BATCH * _SEQ
_NBLK = _VOCAB // 128
_TAIL = _VOCAB - _NBLK * 128
_BPW = _NBLK // _NW
_BREM = _NBLK - _BPW * _NW

_NBH = _BATCH // 128
_NJ = _SEQ * _NBH
_JPW = _NJ // _NW


def _conv_body(tabT_hbm, tailp_hbm, scr_hbm,
               in_v, out_v, r0, r1, w0, w1):
    rsems = (r0, r1)
    wsems = (w0, w1)
    wid = lax.axis_index("s") * 2 + lax.axis_index("c")
    nblk = _BPW + (wid < _BREM).astype(jnp.int32)
    start = wid * _BPW + jnp.minimum(wid, _BREM)

    iota = lax.iota(jnp.int32, 16)

    def read_parts(i, b):
        blk = start + i
        return (tabT_hbm.at[:, pl.ds(blk * 128, 128)], in_v.at[b], rsems[b])

    def start_read(i, b):
        src, dst, sem = read_parts(i, b)
        pltpu.async_copy(src, dst, sem)

    def drain_read(i, b):
        src, dst, sem = read_parts(i, b)
        pltpu.make_async_copy(src, dst, sem).wait()

    def write_parts(i, b):
        blk = start + i
        return (out_v.at[b], scr_hbm.at[pl.ds(blk * 128, 128)], wsems[b])

    def start_write(i, b):
        src, dst, sem = write_parts(i, b)
        pltpu.async_copy(src, dst, sem)

    def drain_write(i, b):
        src, dst, sem = write_parts(i, b)
        pltpu.make_async_copy(src, dst, sem).wait()

    def transpose_scale(b):
        src = in_v.at[b]
        dst = out_v.at[b]

        def row_fn(rr, _):
            rsp = jnp.full((16,), rr, jnp.int32)
            for g in range(_D // 16):
                v = plsc.load_gather(src, [iota + 16 * g, rsp])
                dst[rr, pl.ds(16 * g, 16)] = v * _SCALE
            return 0

        lax.fori_loop(0, 128, row_fn, 0, unroll=8)

    start_read(0, 0)

    def pair(k, _):
        for u in range(2):
            i = k * 2 + u

            @pl.when(i < nblk)
            def _():
                drain_read(i, u)

                @pl.when(i + 1 < nblk)
                def _():
                    start_read(i + 1, 1 - u)

                @pl.when(i >= 2)
                def _():
                    drain_write(i - 2, u)

                transpose_scale(u)
                start_write(i, u)

        return 0

    lax.fori_loop(0, (_BPW + 2) // 2, pair, 0)

    drain_write(0, 0)
    drain_write(0, 1)

    @pl.when(wid == _NW - 1)
    def _():
        pltpu.sync_copy(tailp_hbm, out_v.at[0, pl.ds(0, _TAIL)])
        pltpu.sync_copy(out_v.at[0, pl.ds(0, _TAIL)],
                        scr_hbm.at[pl.ds(_NBLK * 128, _TAIL)])


_convert = functools.partial(
    pl.kernel,
    mesh=plsc.VectorSubcoreMesh(core_axis_name="c", subcore_axis_name="s"),
    compiler_params=pltpu.CompilerParams(needs_layout_passes=False),
    out_type=jax.ShapeDtypeStruct((_VOCAB, _DP), jnp.float32),
    scratch_types=[
        pltpu.VMEM((2, _D, 128), jnp.float32),
        pltpu.VMEM((2, 128, _DP), jnp.float32),
        pltpu.SemaphoreType.DMA,
        pltpu.SemaphoreType.DMA,
        pltpu.SemaphoreType.DMA,
        pltpu.SemaphoreType.DMA,
    ],
)(_conv_body)


def _gather_body(xT_hbm, scr_hbm, out_hbm, idx_v, rows_v, blk_v,
                 i0, i1, i2, g0, g1, g2, w0, w1):
    isems = (i0, i1, i2)
    gsems = (g0, g1, g2)
    wsems = (w0, w1)
    wid = lax.axis_index("s") * 2 + lax.axis_index("c")
    j0 = wid * _JPW

    iota = lax.iota(jnp.int32, 16)

    def sb(t):
        j = j0 + t
        return j // _NBH, j % _NBH

    def idx_parts(t, b):
        s, bh = sb(t)
        return (xT_hbm.at[s, pl.ds(bh * 128, 128)], idx_v.at[b], isems[b])

    def start_idx(t, b):
        src, dst, sem = idx_parts(t, b)
        pltpu.async_copy(src, dst, sem)

    def drain_idx(t, b):
        src, dst, sem = idx_parts(t, b)
        pltpu.make_async_copy(src, dst, sem).wait()

    def gather_parts(t, b):
        return (scr_hbm.at[idx_v.at[b]], rows_v.at[b], gsems[b])

    def start_gather(t, b):
        src, dst, sem = gather_parts(t, b)
        pltpu.async_copy(src, dst, sem)

    def drain_gather(t, b):
        src, dst, sem = gather_parts(t, b)
        pltpu.make_async_copy(src, dst, sem).wait()

    def transpose(b, ob):
        src = rows_v.at[b]
        dst = blk_v.at[ob]

        def d_fn(d, _):
            dhi = d // 8
            dlo = d % 8
            dsp = jnp.full((16,), d, jnp.int32)
            for bg in range(8):
                v = plsc.load_gather(src, [iota + 16 * bg, dsp])
                dst[dhi, dlo, pl.ds(16 * bg, 16)] = v
            return 0

        lax.fori_loop(0, _D, d_fn, 0, unroll=4)

    def write_parts(t, ob, dhi):
        s, bh = sb(t)
        return (blk_v.at[ob, dhi], out_hbm.at[s, dhi, bh], wsems[ob])

    def start_write(t, ob):
        for dhi in range(8):
            src, dst, sem = write_parts(t, ob, dhi)
            pltpu.async_copy(src, dst, sem)

    def drain_write(t, ob):
        for dhi in range(8):
            src, dst, sem = write_parts(t, ob, dhi)
            pltpu.make_async_copy(src, dst, sem).wait()

    start_idx(0, 0)
    start_idx(1, 1)
    drain_idx(0, 0)
    start_gather(0, 0)

    def group_fn(k, _):
        for u in range(6):
            t = k * 6 + u
            b = u % 3
            ob = u % 2

            @pl.when(t < _JPW)
            def _():
                @pl.when(t + 2 < _JPW)
                def _():
                    start_idx(t + 2, (u + 2) % 3)

                @pl.when(t + 1 < _JPW)
                def _():
                    drain_idx(t + 1, (u + 1) % 3)
                    start_gather(t + 1, (u + 1) % 3)

                drain_gather(t, b)

                @pl.when(t >= 2)
                def _():
                    drain_write(t - 2, ob)

                transpose(b, ob)
                start_write(t, ob)

        return 0

    lax.fori_loop(0, (_JPW + 5) // 6, group_fn, 0)

    drain_write(0, 0)
    drain_write(0, 1)


_gather = functools.partial(
    pl.kernel,
    mesh=plsc.VectorSubcoreMesh(core_axis_name="c", subcore_axis_name="s"),
    compiler_params=pltpu.CompilerParams(needs_layout_passes=False),
    out_type=jax.ShapeDtypeStruct((_SEQ, 8, _NBH, 8, 128), jnp.float32),
    scratch_types=[
        pltpu.VMEM((3, 128), jnp.int32),
        pltpu.VMEM((3, 128, _DP), jnp.float32),
        pltpu.VMEM((2, 8, 8, 128), jnp.float32),
        pltpu.SemaphoreType.DMA,
        pltpu.SemaphoreType.DMA,
        pltpu.SemaphoreType.DMA,
        pltpu.SemaphoreType.DMA,
        pltpu.SemaphoreType.DMA,
        pltpu.SemaphoreType.DMA,
        pltpu.SemaphoreType.DMA,
        pltpu.SemaphoreType.DMA,
    ],
)(_gather_body)


@jax.jit
def kernel(x, table):
    xT = x.T
    tabT = table.T
    tailp = jnp.pad(table[_NBLK * 128:, :] * _SCALE,
                    ((0, 0), (0, _DP - _D)))
    scr = _convert(tabT, tailp)
    out5 = _gather(xT, scr)
    out = jnp.transpose(out5, (2, 4, 0, 1, 3)).reshape(_BATCH, _SEQ, _D)
    return out


# speedup vs baseline: 2.8496x; 2.8496x over previous
import functools
import math

import jax
import jax.numpy as jnp
from jax import lax
from jax.experimental import pallas as pl
from jax.experimental.pallas import tpu as pltpu
from jax.experimental.pallas import tpu_sc as plsc

_VOCAB = 1000000
_D = 64
_DP = 128
_BATCH = 4096
_SEQ = 200
_SCALE = math.sqrt(_D)

_NW = 32
_B_TOT = _BATCH * _SEQ
_PER_W = _B_TOT // _NW
_CHUNK = 128
_NCHUNK = _PER_W // _CHUNK
_NBUF = 5


def _body(x_hbm, tab_hbm, out_hbm, idx_v, rows_v, s0, s1, s2, s3, s4):
    sems = (s0, s1, s2, s3, s4)
    wid = lax.axis_index("s") * 2 + lax.axis_index("c")
    base = wid * _PER_W

    pltpu.sync_copy(x_hbm.at[pl.ds(base, _PER_W)], idx_v)

    def gather_parts(c, b):
        isl = idx_v.at[pl.ds(c * _CHUNK, _CHUNK)]
        return tab_hbm.at[isl], rows_v.at[b], sems[b]

    def start_gather(c, b):
        src, dst, sem = gather_parts(c, b)
        pltpu.async_copy(src, dst, sem)

    def drain_gather(c, b):
        src, dst, sem = gather_parts(c, b)
        pltpu.make_async_copy(src, dst, sem).wait()

    def scale(b):
        rb = rows_v.at[b]

        def row_fn(i, _):
            for j in range(_D // 16):
                sl = pl.ds(16 * j, 16)
                rb[i, sl] = rb[i, sl] * _SCALE
            return 0

        lax.fori_loop(0, _CHUNK, row_fn, 0, unroll=4)

    def write(c, b):
        pltpu.sync_copy(rows_v.at[b],
                        out_hbm.at[pl.ds(base + c * _CHUNK, _CHUNK)])

    for c in range(_NBUF - 1):
        start_gather(c, c)

    def group_fn(k, _):
        for u in range(_NBUF):
            c = k * _NBUF + u
            drain_gather(c, u)

            @pl.when(c + _NBUF - 1 < _NCHUNK)
            def _():
                start_gather(c + _NBUF - 1, (u + _NBUF - 1) % _NBUF)

            scale(u)
            write(c, u)
        return 0

    lax.fori_loop(0, _NCHUNK // _NBUF, group_fn, 0)


_emb = functools.partial(
    pl.kernel,
    mesh=plsc.VectorSubcoreMesh(core_axis_name="c", subcore_axis_name="s"),
    out_type=jax.ShapeDtypeStruct((_B_TOT, _DP), jnp.float32),
    scratch_types=[
        pltpu.VMEM((_PER_W,), jnp.int32),
        pltpu.VMEM((_NBUF, _CHUNK, _DP), jnp.float32),
        pltpu.SemaphoreType.DMA,
        pltpu.SemaphoreType.DMA,
        pltpu.SemaphoreType.DMA,
        pltpu.SemaphoreType.DMA,
        pltpu.SemaphoreType.DMA,
    ],
)(_body)


@jax.jit
def kernel(x, table):
    xf = x.reshape(_B_TOT)
    table_p = jnp.pad(table, ((0, 0), (0, _DP - _D)))
    out = _emb(xf, table_p)
    return out[:, :_D].reshape(_BATCH, _SEQ, _D)
